# baseline (speedup 1.0000x reference)
import functools

import jax
import jax.numpy as jnp
from jax import lax
from jax.experimental import pallas as pl
from jax.experimental.pallas import tpu as pltpu
from jax.experimental.pallas import tpu_sc as plsc

NUM_CORES = 2
NUM_SUBCORES = 16
NW = NUM_CORES * NUM_SUBCORES
LANES = 16
SEQ = 16384
B_PER_W = SEQ // NW
EMB = 64
PAIR = 2 * EMB
NTAGS = 1000
NWORDS_HALF = 500000
NCHUNK = 4
CHUNK = B_PER_W // NCHUNK
GROUPS_PER_CHUNK = CHUNK // LANES


def _sc_gather_sum(nwords_i32, table_pairs):
    mesh = plsc.VectorSubcoreMesh(core_axis_name="c", subcore_axis_name="s")

    @functools.partial(
        pl.kernel,
        out_type=jax.ShapeDtypeStruct((NW, EMB), jnp.float32),
        mesh=mesh,
        scratch_types=[
            pltpu.VMEM((B_PER_W,), jnp.int32),
            pltpu.VMEM((B_PER_W,), jnp.int32),
            pltpu.VMEM((B_PER_W,), jnp.int32),
            pltpu.VMEM((B_PER_W, PAIR), jnp.float32),
            pltpu.VMEM((EMB,), jnp.float32),
            pltpu.SemaphoreType.DMA,
        ],
        compiler_params=pltpu.CompilerParams(needs_layout_passes=False),
    )
    def k(idx_hbm, table_hbm, out_hbm, idx_v, idx2_v, p64_v, rows_v, acc_v, sem):
        wid = lax.axis_index("s") * NUM_CORES + lax.axis_index("c")
        base = wid * B_PER_W
        pltpu.sync_copy(idx_hbm.at[pl.ds(base, B_PER_W)], idx_v)

        copies = []
        for j in range(NCHUNK):
            for g in range(GROUPS_PER_CHUNK):
                off = j * CHUNK + g * LANES
                v = idx_v[pl.ds(off, LANES)]
                idx2_v[pl.ds(off, LANES)] = jax.lax.shift_right_logical(v, 1)
            copies.append(pltpu.async_copy(
                table_hbm.at[idx2_v.at[pl.ds(j * CHUNK, CHUNK)]],
                rows_v.at[pl.ds(j * CHUNK, CHUNK)],
                sem,
            ))
        for off in range(0, B_PER_W, LANES):
            v = idx_v[pl.ds(off, LANES)]
            p64_v[pl.ds(off, LANES)] = jax.lax.shift_left(v & 1, 6)
        for cp in copies:
            cp.wait()

        col_base = lax.iota(jnp.int32, LANES)

        def body(i, accs):
            i_splat = jnp.full((LANES,), i, jnp.int32)
            p64 = plsc.load_gather(p64_v, [i_splat])
            return tuple(
                accs[e] + plsc.load_gather(
                    rows_v, [i_splat, p64 + (col_base + e * LANES)])
                for e in range(4)
            )

        accs = lax.fori_loop(
            0, B_PER_W, body,
            tuple(jnp.zeros((LANES,), jnp.float32) for _ in range(4)),
        )
        for e in range(4):
            acc_v[pl.ds(e * LANES, LANES)] = accs[e]
        pltpu.sync_copy(acc_v, out_hbm.at[wid])

    return k(nwords_i32, table_pairs)


def _tc_head(partials, lin_weight, lin_bias2d):
    def body(p_ref, w_ref, b_ref, o_ref):
        s = jnp.sum(p_ref[...], axis=0, keepdims=True)
        o_ref[...] = (
            lax.dot_general(
                s, w_ref[...],
                dimension_numbers=(((1,), (1,)), ((), ())),
                preferred_element_type=jnp.float32,
            )
            + b_ref[...]
        )

    return pl.pallas_call(
        body,
        out_shape=jax.ShapeDtypeStruct((1, NTAGS), jnp.float32),
    )(partials, lin_weight, lin_bias2d)


def kernel(nwords, emb_weight, lin_weight, lin_bias):
    idx = nwords.astype(jnp.int32)
    table_pairs = emb_weight.reshape(NWORDS_HALF, PAIR)
    partials = _sc_gather_sum(idx, table_pairs)
    return _tc_head(partials, lin_weight, lin_bias.reshape(1, NTAGS))


# speedup vs baseline: 4.6296x; 4.6296x over previous
import functools

import jax
import jax.numpy as jnp
from jax import lax
from jax.experimental import pallas as pl
from jax.experimental.pallas import tpu as pltpu
from jax.experimental.pallas import tpu_sc as plsc

NUM_CORES = 2
NUM_SUBCORES = 16
NTILES = NUM_CORES * NUM_SUBCORES
LANES = 16
SEQ = 16384
NWORDS_TOTAL = 1000000
EMB = 64
NTAGS = 1000
HIST = 1 << 20
HIST_PER_TILE = HIST // NTILES
ZERO_UNROLL = 8
COUNT_UNROLL = 4
BLK = 16384
NBLK = -(-NWORDS_TOTAL // BLK)


def _sc_histogram(nwords_i32):
    mesh = plsc.VectorSubcoreMesh(core_axis_name="c", subcore_axis_name="s")

    @functools.partial(
        pl.kernel,
        out_type=jax.ShapeDtypeStruct((HIST,), jnp.float32),
        mesh=mesh,
        scratch_types=[
            pltpu.VMEM((SEQ,), jnp.int32),
            pltpu.VMEM((HIST_PER_TILE,), jnp.float32),
            pltpu.SemaphoreType.DMA,
        ],
        compiler_params=pltpu.CompilerParams(needs_layout_passes=False),
    )
    def k(idx_hbm, out_hbm, idx_v, hist_v, sem):
        wid = lax.axis_index("s") * NUM_CORES + lax.axis_index("c")
        cp = pltpu.async_copy(idx_hbm, idx_v, sem)

        zeros16 = jnp.zeros((LANES,), jnp.float32)

        def zero_body(i, carry):
            for u in range(ZERO_UNROLL):
                hist_v[pl.ds((i * ZERO_UNROLL + u) * LANES, LANES)] = zeros16
            return carry

        lax.fori_loop(0, HIST_PER_TILE // (LANES * ZERO_UNROLL), zero_body, 0)
        cp.wait()

        base_w = wid * HIST_PER_TILE
        ones16 = jnp.ones((LANES,), jnp.float32)

        def count_body(g, carry):
            for u in range(COUNT_UNROLL):
                rel = idx_v[pl.ds((g * COUNT_UNROLL + u) * LANES, LANES)] - base_w
                mask = (rel >= 0) & (rel < HIST_PER_TILE)
                plsc.addupdate_scatter(hist_v, [rel], ones16, mask=mask)
            return carry

        lax.fori_loop(0, SEQ // (LANES * COUNT_UNROLL), count_body, 0)
        pltpu.sync_copy(hist_v, out_hbm.at[pl.ds(base_w, HIST_PER_TILE)])

    return k(nwords_i32)


def _tc_matvec_head(table_t, hist, lin_weight, lin_bias2d):
    def body(t_ref, h_ref, w_ref, b_ref, o_ref, acc_ref):
        step = pl.program_id(0)

        @pl.when(step == 0)
        def _():
            acc_ref[...] = jnp.zeros((1, EMB), jnp.float32)

        h = h_ref[...]
        t = t_ref[...]

        @pl.when(step == NBLK - 1)
        def _():
            lane = lax.broadcasted_iota(jnp.int32, (1, BLK), 1)
            limit = NWORDS_TOTAL - (NBLK - 1) * BLK
            acc_ref[...] += lax.dot_general(
                h, jnp.where(lane < limit, t, 0.0),
                dimension_numbers=(((1,), (1,)), ((), ())),
                preferred_element_type=jnp.float32,
            )

        @pl.when(step < NBLK - 1)
        def _():
            acc_ref[...] += lax.dot_general(
                h, t,
                dimension_numbers=(((1,), (1,)), ((), ())),
                preferred_element_type=jnp.float32,
            )

        @pl.when(step == NBLK - 1)
        def _():
            o_ref[...] = (
                lax.dot_general(
                    acc_ref[...], w_ref[...],
                    dimension_numbers=(((1,), (1,)), ((), ())),
                    preferred_element_type=jnp.float32,
                )
                + b_ref[...]
            )

    return pl.pallas_call(
        body,
        grid=(NBLK,),
        in_specs=[
            pl.BlockSpec((EMB, BLK), lambda k: (0, k)),
            pl.BlockSpec((1, BLK), lambda k: (0, k)),
            pl.BlockSpec((NTAGS, EMB), lambda k: (0, 0)),
            pl.BlockSpec((1, NTAGS), lambda k: (0, 0)),
        ],
        out_specs=pl.BlockSpec((1, NTAGS), lambda k: (0, 0)),
        scratch_shapes=[pltpu.VMEM((1, EMB), jnp.float32)],
        out_shape=jax.ShapeDtypeStruct((1, NTAGS), jnp.float32),
    )(table_t, hist, lin_weight, lin_bias2d)


def kernel(nwords, emb_weight, lin_weight, lin_bias):
    idx = nwords.astype(jnp.int32)
    table_t = emb_weight.T
    hist = _sc_histogram(idx).reshape(1, HIST)
    return _tc_matvec_head(table_t, hist, lin_weight,
                           lin_bias.reshape(1, NTAGS))


# speedup vs baseline: 5.3296x; 1.1512x over previous
import functools

import jax
import jax.numpy as jnp
from jax import lax
from jax.experimental import pallas as pl
from jax.experimental.pallas import tpu as pltpu
from jax.experimental.pallas import tpu_sc as plsc

NUM_CORES = 2
NUM_SUBCORES = 16
NTILES = NUM_CORES * NUM_SUBCORES
LANES = 16
SEQ = 16384
NWORDS_TOTAL = 1000000
EMB = 64
NTAGS = 1000
HIST = 1 << 20
HIST_PER_TILE = HIST // NTILES
ZERO_UNROLL = 8
COUNT_UNROLL = 4
BLK = 32768
NBLK = -(-NWORDS_TOTAL // BLK)


def _sc_histogram(nwords_i32):
    mesh = plsc.VectorSubcoreMesh(core_axis_name="c", subcore_axis_name="s")

    @functools.partial(
        pl.kernel,
        out_type=jax.ShapeDtypeStruct((HIST,), jnp.float32),
        mesh=mesh,
        scratch_types=[
            pltpu.VMEM((SEQ,), jnp.int32),
            pltpu.VMEM((HIST_PER_TILE,), jnp.float32),
            pltpu.SemaphoreType.DMA,
        ],
        compiler_params=pltpu.CompilerParams(needs_layout_passes=False),
    )
    def k(idx_hbm, out_hbm, idx_v, hist_v, sem):
        wid = lax.axis_index("s") * NUM_CORES + lax.axis_index("c")
        cp = pltpu.async_copy(idx_hbm, idx_v, sem)

        zeros16 = jnp.zeros((LANES,), jnp.float32)

        def zero_body(i, carry):
            for u in range(ZERO_UNROLL):
                hist_v[pl.ds((i * ZERO_UNROLL + u) * LANES, LANES)] = zeros16
            return carry

        lax.fori_loop(0, HIST_PER_TILE // (LANES * ZERO_UNROLL), zero_body, 0)
        cp.wait()

        base_w = wid * HIST_PER_TILE
        ones16 = jnp.ones((LANES,), jnp.float32)

        def count_body(g, carry):
            for u in range(COUNT_UNROLL):
                rel = idx_v[pl.ds((g * COUNT_UNROLL + u) * LANES, LANES)] - base_w
                mask = (rel >= 0) & (rel < HIST_PER_TILE)
                plsc.addupdate_scatter(hist_v, [rel], ones16, mask=mask)
            return carry

        lax.fori_loop(0, SEQ // (LANES * COUNT_UNROLL), count_body, 0)
        pltpu.sync_copy(hist_v, out_hbm.at[pl.ds(base_w, HIST_PER_TILE)])

    return k(nwords_i32)


def _tc_matvec_head(table_t, hist, lin_weight, lin_bias2d):
    def body(t_ref, h_ref, w_ref, b_ref, o_ref, acc_ref):
        step = pl.program_id(0)

        @pl.when(step == 0)
        def _():
            acc_ref[...] = jnp.zeros((1, EMB), jnp.float32)

        h = h_ref[...]
        t = t_ref[...]

        @pl.when(step == NBLK - 1)
        def _():
            lane = lax.broadcasted_iota(jnp.int32, (1, BLK), 1)
            limit = NWORDS_TOTAL - (NBLK - 1) * BLK
            acc_ref[...] += lax.dot_general(
                h, jnp.where(lane < limit, t, 0.0),
                dimension_numbers=(((1,), (1,)), ((), ())),
                preferred_element_type=jnp.float32,
            )

        @pl.when(step < NBLK - 1)
        def _():
            acc_ref[...] += lax.dot_general(
                h, t,
                dimension_numbers=(((1,), (1,)), ((), ())),
                preferred_element_type=jnp.float32,
            )

        @pl.when(step == NBLK - 1)
        def _():
            o_ref[...] = (
                lax.dot_general(
                    acc_ref[...], w_ref[...],
                    dimension_numbers=(((1,), (1,)), ((), ())),
                    preferred_element_type=jnp.float32,
                )
                + b_ref[...]
            )

    return pl.pallas_call(
        body,
        grid=(NBLK,),
        in_specs=[
            pl.BlockSpec((EMB, BLK), lambda k: (0, k)),
            pl.BlockSpec((1, BLK), lambda k: (0, k)),
            pl.BlockSpec((NTAGS, EMB), lambda k: (0, 0)),
            pl.BlockSpec((1, NTAGS), lambda k: (0, 0)),
        ],
        out_specs=pl.BlockSpec((1, NTAGS), lambda k: (0, 0)),
        scratch_shapes=[pltpu.VMEM((1, EMB), jnp.float32)],
        out_shape=jax.ShapeDtypeStruct((1, NTAGS), jnp.float32),
    )(table_t, hist, lin_weight, lin_bias2d)


def kernel(nwords, emb_weight, lin_weight, lin_bias):
    idx = nwords.astype(jnp.int32)
    table_t = emb_weight.T
    hist = _sc_histogram(idx).reshape(1, HIST)
    return _tc_matvec_head(table_t, hist, lin_weight,
                           lin_bias.reshape(1, NTAGS))


# speedup vs baseline: 5.4836x; 1.0289x over previous
import functools

import jax
import jax.numpy as jnp
from jax import lax
from jax.experimental import pallas as pl
from jax.experimental.pallas import tpu as pltpu
from jax.experimental.pallas import tpu_sc as plsc

NUM_CORES = 2
NUM_SUBCORES = 16
NTILES = NUM_CORES * NUM_SUBCORES
LANES = 16
SEQ = 16384
NWORDS_TOTAL = 1000000
EMB = 64
NTAGS = 1000
HIST = 1 << 20
HIST_PER_TILE = HIST // NTILES
ZERO_UNROLL = 8
COUNT_UNROLL = 4
BLK = 65536
NBLK = -(-NWORDS_TOTAL // BLK)


def _sc_histogram(nwords_i32):
    mesh = plsc.VectorSubcoreMesh(core_axis_name="c", subcore_axis_name="s")

    @functools.partial(
        pl.kernel,
        out_type=jax.ShapeDtypeStruct((HIST,), jnp.float32),
        mesh=mesh,
        scratch_types=[
            pltpu.VMEM((SEQ,), jnp.int32),
            pltpu.VMEM((HIST_PER_TILE,), jnp.float32),
            pltpu.SemaphoreType.DMA,
        ],
        compiler_params=pltpu.CompilerParams(needs_layout_passes=False),
    )
    def k(idx_hbm, out_hbm, idx_v, hist_v, sem):
        wid = lax.axis_index("s") * NUM_CORES + lax.axis_index("c")
        cp = pltpu.async_copy(idx_hbm, idx_v, sem)

        zeros16 = jnp.zeros((LANES,), jnp.float32)

        def zero_body(i, carry):
            for u in range(ZERO_UNROLL):
                hist_v[pl.ds((i * ZERO_UNROLL + u) * LANES, LANES)] = zeros16
            return carry

        lax.fori_loop(0, HIST_PER_TILE // (LANES * ZERO_UNROLL), zero_body, 0)
        cp.wait()

        base_w = wid * HIST_PER_TILE
        ones16 = jnp.ones((LANES,), jnp.float32)

        def count_body(g, carry):
            for u in range(COUNT_UNROLL):
                rel = idx_v[pl.ds((g * COUNT_UNROLL + u) * LANES, LANES)] - base_w
                mask = (rel >= 0) & (rel < HIST_PER_TILE)
                plsc.addupdate_scatter(hist_v, [rel], ones16, mask=mask)
            return carry

        lax.fori_loop(0, SEQ // (LANES * COUNT_UNROLL), count_body, 0)
        pltpu.sync_copy(hist_v, out_hbm.at[pl.ds(base_w, HIST_PER_TILE)])

    return k(nwords_i32)


def _tc_matvec_head(table_t, hist, lin_weight, lin_bias2d):
    def body(t_ref, h_ref, w_ref, b_ref, o_ref, acc_ref):
        step = pl.program_id(0)

        @pl.when(step == 0)
        def _():
            acc_ref[...] = jnp.zeros((1, EMB), jnp.float32)

        h = h_ref[...]
        t = t_ref[...]

        @pl.when(step == NBLK - 1)
        def _():
            lane = lax.broadcasted_iota(jnp.int32, (1, BLK), 1)
            limit = NWORDS_TOTAL - (NBLK - 1) * BLK
            acc_ref[...] += lax.dot_general(
                h, jnp.where(lane < limit, t, 0.0),
                dimension_numbers=(((1,), (1,)), ((), ())),
                preferred_element_type=jnp.float32,
            )

        @pl.when(step < NBLK - 1)
        def _():
            acc_ref[...] += lax.dot_general(
                h, t,
                dimension_numbers=(((1,), (1,)), ((), ())),
                preferred_element_type=jnp.float32,
            )

        @pl.when(step == NBLK - 1)
        def _():
            o_ref[...] = (
                lax.dot_general(
                    acc_ref[...], w_ref[...],
                    dimension_numbers=(((1,), (1,)), ((), ())),
                    preferred_element_type=jnp.float32,
                )
                + b_ref[...]
            )

    return pl.pallas_call(
        body,
        grid=(NBLK,),
        in_specs=[
            pl.BlockSpec((EMB, BLK), lambda k: (0, k)),
            pl.BlockSpec((1, BLK), lambda k: (0, k)),
            pl.BlockSpec((NTAGS, EMB), lambda k: (0, 0)),
            pl.BlockSpec((1, NTAGS), lambda k: (0, 0)),
        ],
        out_specs=pl.BlockSpec((1, NTAGS), lambda k: (0, 0)),
        scratch_shapes=[pltpu.VMEM((1, EMB), jnp.float32)],
        out_shape=jax.ShapeDtypeStruct((1, NTAGS), jnp.float32),
    )(table_t, hist, lin_weight, lin_bias2d)


def kernel(nwords, emb_weight, lin_weight, lin_bias):
    idx = nwords.astype(jnp.int32)
    table_t = emb_weight.T
    hist = _sc_histogram(idx).reshape(1, HIST)
    return _tc_matvec_head(table_t, hist, lin_weight,
                           lin_bias.reshape(1, NTAGS))
